# baseline (speedup 1.0000x reference)
import functools

import jax
import jax.numpy as jnp
from jax import lax
from jax.experimental import pallas as pl
from jax.experimental.pallas import tpu as pltpu
from jax.experimental.pallas import tpu_sc as plsc

NC = 2
NS = 16
NW = NC * NS
L = 16

CHUNK = 16384
_MAGIC = 12582912.0


def _pwl_body(x_hbm, bp_hbm, ss_hbm, sv_hbm, is_hbm, iv_hbm, out_hbm,
              in0, in1, out0, out1, bp_v, ss_v, sv_v, is_v, iv_v,
              qs_seg, qi_seg, qs_lut, qi_lut,
              sem_i0, sem_i1, sem_o0, sem_o1, n_chunks: int, per_w: int):
    wid = lax.axis_index("s") * NC + lax.axis_index("c")
    base = wid * per_w

    pltpu.sync_copy(bp_hbm, bp_v)
    pltpu.sync_copy(ss_hbm, ss_v)
    pltpu.sync_copy(sv_hbm, sv_v)
    pltpu.sync_copy(is_hbm, is_v)
    pltpu.sync_copy(iv_hbm, iv_v)

    def exp2_int(e_f32):
        e = e_f32.astype(jnp.int32)
        return lax.bitcast_convert_type(jnp.left_shift(e + 127, 23), jnp.float32)

    for h in range(2):
        sl = pl.ds(h * L, L)
        qs_seg[sl] = sv_v[sl] * exp2_int(ss_v[sl] - 7.0)
        qi_seg[sl] = iv_v[sl] * exp2_int(is_v[sl] - 7.0)

    bp_vec = bp_v[...]
    lanes = lax.iota(jnp.int32, L)
    bps = [jnp.sum(jnp.where(lanes == j, bp_vec, 0.0)) for j in range(16)]
    one = jnp.full((L,), 1, jnp.int32)
    zero = jnp.full((L,), 0, jnp.int32)
    for v in range(256 // L):
        fip_f = (lanes + (v * L - 128)).astype(jnp.float32)
        idx = zero
        for j in range(16):
            idx = idx + jnp.where(fip_f >= bps[j], one, zero)
        sl = pl.ds(v * L, L)
        qs_lut[sl] = plsc.load_gather(qs_seg, [idx])
        qi_lut[sl] = plsc.load_gather(qi_seg, [idx])

    ins = (in0, in1)
    outs = (out0, out1)
    sem_i = (sem_i0, sem_i1)
    sem_o = (sem_o0, sem_o1)

    def in_slice(k):
        return x_hbm.at[pl.ds(base + k * CHUNK, CHUNK)]

    def out_slice(k):
        return out_hbm.at[pl.ds(base + k * CHUNK, CHUNK)]

    for b in range(2):
        pltpu.async_copy(in_slice(b), ins[b], sem_i[b])

    def compute(in_b, out_b):
        @plsc.parallel_loop(0, CHUNK, step=L, unroll=12)
        def _(off):
            xv = in_b[pl.ds(off, L)]
            xi32 = lax.bitcast_convert_type(xv, jnp.int32)
            m = lax.bitwise_and(xi32, jnp.int32(0x7F800000))
            mc = jnp.clip(m, jnp.int32(0x3F000000), jnp.int32(0x42800000))
            scale = lax.bitcast_convert_type(jnp.int32(-2113929216) - mc,
                                             jnp.float32)
            inv = lax.bitcast_convert_type(mc - jnp.int32(0x03000000),
                                           jnp.float32)
            v = xv * scale
            r = (v + _MAGIC) - _MAGIC
            r = jnp.clip(r, -128.0, 127.0)
            scale2 = lax.bitcast_convert_type(
                jnp.minimum(mc - jnp.int32(0x01000000), jnp.int32(0x3F800000)),
                jnp.float32)
            key = (r * scale2 + 128.0).astype(jnp.int32)
            qs = plsc.load_gather(qs_lut, [key])
            qi = plsc.load_gather(qi_lut, [key])
            out_b[pl.ds(off, L)] = (r * inv) * qs + qi

    @pl.loop(0, n_chunks, step=2)
    def _(g):
        for b in range(2):
            k = g + b
            pltpu.make_async_copy(in_slice(k), ins[b], sem_i[b]).wait()

            @pl.when(k >= 2)
            def _():
                pltpu.make_async_copy(outs[b], out_slice(k - 2), sem_o[b]).wait()

            compute(ins[b], outs[b])
            pltpu.async_copy(outs[b], out_slice(k), sem_o[b])

            @pl.when(k + 2 < n_chunks)
            def _():
                pltpu.async_copy(in_slice(k + 2), ins[b], sem_i[b])

    for b in range(2):
        pltpu.make_async_copy(outs[b], out_slice(n_chunks - 2 + b), sem_o[b]).wait()


def kernel(x, breakpoint_fip, slope_scale, slope_value, intercept_scale,
           intercept_value):
    orig_shape = x.shape
    rows = x.shape[0] * x.shape[1]
    cols = x.shape[2]
    xf = (x.reshape(rows // 8, 8, cols // 128, 128)
           .transpose(0, 2, 1, 3)
           .reshape(-1))
    n = xf.shape[0]
    assert n % (NW * CHUNK) == 0
    per_w = n // NW
    n_chunks = per_w // CHUNK
    assert n_chunks % 2 == 0

    pad = lambda t: jnp.zeros((2 * L,), jnp.float32).at[:t.shape[0]].set(t)
    ss = pad(slope_scale)
    sv = pad(slope_value)
    isc = pad(intercept_scale)
    iv = pad(intercept_value)

    mesh = plsc.VectorSubcoreMesh(core_axis_name="c", subcore_axis_name="s")
    f = pl.kernel(
        functools.partial(_pwl_body, n_chunks=n_chunks, per_w=per_w),
        out_type=jax.ShapeDtypeStruct((n,), jnp.float32),
        mesh=mesh,
        compiler_params=pltpu.CompilerParams(needs_layout_passes=False),
        scratch_types=[
            pltpu.VMEM((CHUNK,), jnp.float32),
            pltpu.VMEM((CHUNK,), jnp.float32),
            pltpu.VMEM((CHUNK,), jnp.float32),
            pltpu.VMEM((CHUNK,), jnp.float32),
            pltpu.VMEM((L,), jnp.float32),
            pltpu.VMEM((2 * L,), jnp.float32),
            pltpu.VMEM((2 * L,), jnp.float32),
            pltpu.VMEM((2 * L,), jnp.float32),
            pltpu.VMEM((2 * L,), jnp.float32),
            pltpu.VMEM((2 * L,), jnp.float32),
            pltpu.VMEM((2 * L,), jnp.float32),
            pltpu.VMEM((256,), jnp.float32),
            pltpu.VMEM((256,), jnp.float32),
            pltpu.SemaphoreType.DMA,
            pltpu.SemaphoreType.DMA,
            pltpu.SemaphoreType.DMA,
            pltpu.SemaphoreType.DMA,
        ],
    )
    out = f(xf, breakpoint_fip, ss, sv, isc, iv)
    return (out.reshape(rows // 8, cols // 128, 8, 128)
               .transpose(0, 2, 1, 3)
               .reshape(orig_shape))


# speedup vs baseline: 1.0949x; 1.0949x over previous
import functools

import jax
import jax.numpy as jnp
from jax import lax
from jax.experimental import pallas as pl
from jax.experimental.pallas import tpu as pltpu
from jax.experimental.pallas import tpu_sc as plsc

NC = 2
NS = 16
NW = NC * NS
L = 16

CHUNK = 16384
_MAGIC = 12582912.0


def _pwl_body(x_hbm, bp_hbm, ss_hbm, sv_hbm, is_hbm, iv_hbm, out_hbm,
              in0, in1, out0, out1, bp_v, ss_v, sv_v, is_v, iv_v,
              qs_seg, qi_seg, qs_lut, qi_lut,
              sem_i0, sem_i1, sem_o0, sem_o1, n_chunks: int, per_w: int):
    wid = lax.axis_index("s") * NC + lax.axis_index("c")
    base = wid * per_w

    pltpu.sync_copy(bp_hbm, bp_v)
    pltpu.sync_copy(ss_hbm, ss_v)
    pltpu.sync_copy(sv_hbm, sv_v)
    pltpu.sync_copy(is_hbm, is_v)
    pltpu.sync_copy(iv_hbm, iv_v)

    def exp2_int(e_f32):
        e = e_f32.astype(jnp.int32)
        return lax.bitcast_convert_type(jnp.left_shift(e + 127, 23), jnp.float32)

    for h in range(2):
        sl = pl.ds(h * L, L)
        qs_seg[sl] = sv_v[sl] * exp2_int(ss_v[sl] - 7.0)
        qi_seg[sl] = iv_v[sl] * exp2_int(is_v[sl] - 7.0)

    bp_vec = bp_v[...]
    lanes = lax.iota(jnp.int32, L)
    bps = [jnp.sum(jnp.where(lanes == j, bp_vec, 0.0)) for j in range(16)]
    one = jnp.full((L,), 1, jnp.int32)
    zero = jnp.full((L,), 0, jnp.int32)
    for v in range(256 // L):
        fip_f = (lanes + (v * L - 128)).astype(jnp.float32)
        idx = zero
        for j in range(16):
            idx = idx + jnp.where(fip_f >= bps[j], one, zero)
        sl = pl.ds(v * L, L)
        qs_lut[sl] = plsc.load_gather(qs_seg, [idx])
        qi_lut[sl] = plsc.load_gather(qi_seg, [idx])

    ins = (in0, in1)
    outs = (out0, out1)
    sem_i = (sem_i0, sem_i1)
    sem_o = (sem_o0, sem_o1)

    def in_slice(k):
        return x_hbm.at[pl.ds(base + k * CHUNK, CHUNK)]

    def out_slice(k):
        return out_hbm.at[pl.ds(base + k * CHUNK, CHUNK)]

    for b in range(2):
        pltpu.async_copy(in_slice(b), ins[b], sem_i[b])

    def compute(in_b, out_b):
        @plsc.parallel_loop(0, CHUNK, step=L, unroll=8)
        def _(off):
            xv = in_b[pl.ds(off, L)]
            xi32 = lax.bitcast_convert_type(xv, jnp.int32)
            m = lax.bitwise_and(xi32, jnp.int32(0x7F800000))
            mc = jnp.clip(m, jnp.int32(0x3F000000), jnp.int32(0x42800000))
            scale = lax.bitcast_convert_type(jnp.int32(-2113929216) - mc,
                                             jnp.float32)
            inv = lax.bitcast_convert_type(mc - jnp.int32(0x03000000),
                                           jnp.float32)
            v = xv * scale
            r = (v + _MAGIC) - _MAGIC
            r = jnp.clip(r, -128.0, 127.0)
            scale2 = lax.bitcast_convert_type(
                jnp.minimum(mc - jnp.int32(0x01000000), jnp.int32(0x3F800000)),
                jnp.float32)
            key = (r * scale2 + 128.0).astype(jnp.int32)
            qs = plsc.load_gather(qs_lut, [key])
            qi = plsc.load_gather(qi_lut, [key])
            out_b[pl.ds(off, L)] = (r * inv) * qs + qi

    @pl.loop(0, n_chunks, step=2)
    def _(g):
        for b in range(2):
            k = g + b
            pltpu.make_async_copy(in_slice(k), ins[b], sem_i[b]).wait()

            @pl.when(k >= 2)
            def _():
                pltpu.make_async_copy(outs[b], out_slice(k - 2), sem_o[b]).wait()

            compute(ins[b], outs[b])
            pltpu.async_copy(outs[b], out_slice(k), sem_o[b])

            @pl.when(k + 2 < n_chunks)
            def _():
                pltpu.async_copy(in_slice(k + 2), ins[b], sem_i[b])

    for b in range(2):
        pltpu.make_async_copy(outs[b], out_slice(n_chunks - 2 + b), sem_o[b]).wait()


def kernel(x, breakpoint_fip, slope_scale, slope_value, intercept_scale,
           intercept_value):
    orig_shape = x.shape
    rows = x.shape[0] * x.shape[1]
    cols = x.shape[2]
    xf = (x.reshape(rows // 8, 8, cols // 128, 128)
           .transpose(0, 2, 1, 3)
           .reshape(-1))
    n = xf.shape[0]
    assert n % (NW * CHUNK) == 0
    per_w = n // NW
    n_chunks = per_w // CHUNK
    assert n_chunks % 2 == 0

    pad = lambda t: jnp.zeros((2 * L,), jnp.float32).at[:t.shape[0]].set(t)
    ss = pad(slope_scale)
    sv = pad(slope_value)
    isc = pad(intercept_scale)
    iv = pad(intercept_value)

    mesh = plsc.VectorSubcoreMesh(core_axis_name="c", subcore_axis_name="s")
    f = pl.kernel(
        functools.partial(_pwl_body, n_chunks=n_chunks, per_w=per_w),
        out_type=jax.ShapeDtypeStruct((n,), jnp.float32),
        mesh=mesh,
        compiler_params=pltpu.CompilerParams(needs_layout_passes=False),
        scratch_types=[
            pltpu.VMEM((CHUNK,), jnp.float32),
            pltpu.VMEM((CHUNK,), jnp.float32),
            pltpu.VMEM((CHUNK,), jnp.float32),
            pltpu.VMEM((CHUNK,), jnp.float32),
            pltpu.VMEM((L,), jnp.float32),
            pltpu.VMEM((2 * L,), jnp.float32),
            pltpu.VMEM((2 * L,), jnp.float32),
            pltpu.VMEM((2 * L,), jnp.float32),
            pltpu.VMEM((2 * L,), jnp.float32),
            pltpu.VMEM((2 * L,), jnp.float32),
            pltpu.VMEM((2 * L,), jnp.float32),
            pltpu.VMEM((256,), jnp.float32),
            pltpu.VMEM((256,), jnp.float32),
            pltpu.SemaphoreType.DMA,
            pltpu.SemaphoreType.DMA,
            pltpu.SemaphoreType.DMA,
            pltpu.SemaphoreType.DMA,
        ],
    )
    out = f(xf, breakpoint_fip, ss, sv, isc, iv)
    return (out.reshape(rows // 8, cols // 128, 8, 128)
               .transpose(0, 2, 1, 3)
               .reshape(orig_shape))
